# baseline (speedup 1.0000x reference)
import functools

import jax
import jax.numpy as jnp
from jax import lax
from jax.experimental import pallas as pl
from jax.experimental.pallas import tpu as pltpu
from jax.experimental.pallas import tpu_sc as plsc

NUM_TOKENS = 4096 * 200
DIM = 64
NW = 32
PER_W = NUM_TOKENS // NW
CHUNK = 512
N_CHUNKS = PER_W // CHUNK


def _gather_kernel(idx_hbm, table_hbm, out_hbm, idx_v, rows_v, sem):
    wid = lax.axis_index("s") * 2 + lax.axis_index("c")
    base = wid * PER_W

    def body(i, carry):
        off = base + i * CHUNK
        pltpu.sync_copy(idx_hbm.at[pl.ds(off, CHUNK)], idx_v)
        pltpu.async_copy(table_hbm.at[idx_v], rows_v, sem).wait()
        pltpu.sync_copy(rows_v, out_hbm.at[pl.ds(off, CHUNK)])
        return carry

    lax.fori_loop(0, N_CHUNKS, body, 0)


def kernel(token_ids, weight):
    idx_flat = token_ids.reshape(-1).astype(jnp.int32)
    mesh = plsc.VectorSubcoreMesh(core_axis_name="c", subcore_axis_name="s")
    run = functools.partial(
        pl.kernel,
        mesh=mesh,
        compiler_params=pltpu.CompilerParams(use_tc_tiling_on_sc=False),
        out_type=jax.ShapeDtypeStruct((NUM_TOKENS, DIM), jnp.float32),
        scratch_types=[
            pltpu.VMEM((CHUNK,), jnp.int32),
            pltpu.VMEM((CHUNK, DIM), jnp.float32),
            pltpu.SemaphoreType.DMA,
        ],
    )(_gather_kernel)
    out = run(idx_flat, weight)
    return out.reshape(token_ids.shape + (DIM,))


# speedup vs baseline: 1.0496x; 1.0496x over previous
import functools

import jax
import jax.numpy as jnp
from jax import lax
from jax.experimental import pallas as pl
from jax.experimental.pallas import tpu as pltpu
from jax.experimental.pallas import tpu_sc as plsc

NUM_TOKENS = 4096 * 200
DIM = 64
NW = 32
PER_W = NUM_TOKENS // NW
NBUF = 4
CHUNK = 400
N_CHUNKS = PER_W // CHUNK
N_ROUNDS = N_CHUNKS // NBUF


def _gather_kernel(idx_hbm, table_hbm, out_hbm, idx_v, rows_v, *sems):
    gsem = sems[:NBUF]
    wsem = sems[NBUF:]
    wid = lax.axis_index("s") * 2 + lax.axis_index("c")
    base = wid * PER_W

    def stage_and_fire(c, b):
        off = base + c * CHUNK
        pltpu.sync_copy(idx_hbm.at[pl.ds(off, CHUNK)], idx_v.at[b])
        pltpu.async_copy(table_hbm.at[idx_v.at[b]], rows_v.at[b], gsem[b])

    def wait_gather(b):
        pltpu.make_async_copy(
            table_hbm.at[idx_v.at[b]], rows_v.at[b], gsem[b]
        ).wait()

    def fire_writeback(c, b):
        off = base + c * CHUNK
        pltpu.async_copy(rows_v.at[b], out_hbm.at[pl.ds(off, CHUNK)], wsem[b])

    def wait_writeback(b):
        pltpu.make_async_copy(
            rows_v.at[b], out_hbm.at[pl.ds(base, CHUNK)], wsem[b]
        ).wait()

    def round_steps(j, skip_early_wwait=False):
        for b in range(NBUF):
            c = j * NBUF + b
            bf = (b + 2) % NBUF
            if not (skip_early_wwait and b < 2):
                wait_writeback(bf)
            stage_and_fire(c + 2, bf)
            wait_gather(b)
            fire_writeback(c, b)

    stage_and_fire(0, 0)
    stage_and_fire(1, 1)

    round_steps(0, skip_early_wwait=True)

    def body(j, carry):
        round_steps(j)
        return carry

    lax.fori_loop(1, N_ROUNDS - 1, body, 0)

    j = N_ROUNDS - 1
    for b in range(NBUF):
        c = j * NBUF + b
        bf = (b + 2) % NBUF
        if c + 2 < N_CHUNKS:
            wait_writeback(bf)
            stage_and_fire(c + 2, bf)
        wait_gather(b)
        fire_writeback(c, b)
    for b in range(NBUF):
        wait_writeback(b)


def kernel(token_ids, weight):
    idx_flat = token_ids.reshape(-1).astype(jnp.int32)
    mesh = plsc.VectorSubcoreMesh(core_axis_name="c", subcore_axis_name="s")
    run = functools.partial(
        pl.kernel,
        mesh=mesh,
        compiler_params=pltpu.CompilerParams(use_tc_tiling_on_sc=False),
        out_type=jax.ShapeDtypeStruct((NUM_TOKENS, DIM), jnp.float32),
        scratch_types=[
            pltpu.VMEM((NBUF, CHUNK), jnp.int32),
            pltpu.VMEM((NBUF, CHUNK, DIM), jnp.float32),
        ]
        + [pltpu.SemaphoreType.DMA] * (2 * NBUF),
    )(_gather_kernel)
    out = run(idx_flat, weight)
    return out.reshape(token_ids.shape + (DIM,))
